# baseline (speedup 1.0000x reference)
import jax
import jax.numpy as jnp
from jax.experimental import pallas as pl
from jax.experimental.pallas import tpu as pltpu

_ROWS = 128
_COLS = 100000
_BLOCK_C = 2048
_K = 4
_STRIDE = _K * _BLOCK_C
_NB = (_COLS + _STRIDE - 1) // _STRIDE

_U32 = jnp.uint32
_TINY = 1.1754943508222875e-38


def _threefry2x32(x1):
    ks0 = _U32(0)
    ks1 = _U32(42)
    ks2 = _U32(0 ^ 42 ^ 0x1BD11BDA)

    def rotl(x, d):
        return (x << _U32(d)) | (x >> _U32(32 - d))

    def rounds(x0, x1, rots):
        for r in rots:
            x0 = x0 + x1
            x1 = rotl(x1, r)
            x1 = x0 ^ x1
        return x0, x1

    r_even = (13, 15, 26, 6)
    r_odd = (17, 29, 16, 24)
    x1 = x1 + ks1
    x0 = x1
    x1 = x1 ^ rotl(x1, 13)
    x0, x1 = rounds(x0, x1, r_even[1:])
    x0 = x0 + ks1
    x1 = x1 + ks2 + _U32(1)
    x0, x1 = rounds(x0, x1, r_odd)
    x0 = x0 + ks2
    x1 = x1 + ks0 + _U32(2)
    x0, x1 = rounds(x0, x1, r_even)
    x0 = x0 + ks0
    x1 = x1 + ks1 + _U32(3)
    x0, x1 = rounds(x0, x1, r_odd)
    x0 = x0 + ks1
    x1 = x1 + ks2 + _U32(4)
    x0, x1 = rounds(x0, x1, r_even)
    x0 = x0 + ks2
    x1 = x1 + ks0 + _U32(5)
    return x0, x1


def _chain(blk, col0):
    j = col0 + jax.lax.broadcasted_iota(jnp.int32, blk.shape, 1)
    row = jax.lax.broadcasted_iota(jnp.int32, blk.shape, 0)
    n = (row * _COLS + j).astype(_U32)

    r0, r1 = _threefry2x32(n)
    bits = r0 ^ r1

    fb = (bits >> _U32(9)) | _U32(0x3F800000)
    floats = jax.lax.bitcast_convert_type(fb, jnp.float32) - jnp.float32(1.0)
    u = jnp.maximum(_TINY, floats + _TINY)
    g = -jnp.log(-jnp.log(u))

    val = g + blk
    val = jnp.where(j < _COLS, val, jnp.float32(-jnp.inf))
    return val, j


def _comb(a, b):
    va, ia = a
    vb, ib = b
    keep = va >= vb
    return jnp.where(keep, va, vb), jnp.where(keep, ia, ib)


def _sample_kernel(l0, l1, l2, l3, out_ref, acc_ref, idx_ref):
    step = pl.program_id(0)
    base = step * _STRIDE

    pairs = [
        _chain(ref[...], base + k * _BLOCK_C)
        for k, ref in enumerate((l0, l1, l2, l3))
    ]
    v, i = _comb(_comb(pairs[0], pairs[1]), _comb(pairs[2], pairs[3]))

    acc_old = jnp.where(step == 0, jnp.float32(-jnp.inf), acc_ref[...])
    upd = v > acc_old
    acc_ref[...] = jnp.maximum(v, acc_old)
    idx_ref[...] = jnp.where(upd, i, idx_ref[...])

    @pl.when(step == _NB - 1)
    def _():
        acc = acc_ref[...]
        idx = idx_ref[...]
        bmax = jnp.max(acc, axis=1, keepdims=True)
        cand = jnp.where(acc == bmax, idx, jnp.int32(2**31 - 1))
        out_ref[...] = jnp.min(cand, axis=1, keepdims=True)


_LAST_BLOCK = (_COLS + _BLOCK_C - 1) // _BLOCK_C - 1


def _in_spec(k):
    return pl.BlockSpec(
        (_ROWS, _BLOCK_C),
        lambda i, _k=k: (0, jnp.minimum(_K * i + _k, _LAST_BLOCK)),
    )


@jax.jit
def kernel(logits):
    out = pl.pallas_call(
        _sample_kernel,
        grid=(_NB,),
        in_specs=[_in_spec(k) for k in range(_K)],
        out_specs=pl.BlockSpec((_ROWS, 1), lambda i: (0, 0)),
        out_shape=jax.ShapeDtypeStruct((_ROWS, 1), jnp.int32),
        scratch_shapes=[
            pltpu.VMEM((_ROWS, _BLOCK_C), jnp.float32),
            pltpu.VMEM((_ROWS, _BLOCK_C), jnp.int32),
        ],
    )(logits, logits, logits, logits)
    return out.reshape(_ROWS).astype(jnp.int64)


# speedup vs baseline: 1.0317x; 1.0317x over previous
import jax
import jax.numpy as jnp
from jax.experimental import pallas as pl
from jax.experimental.pallas import tpu as pltpu

_ROWS = 128
_COLS = 100000
_BLOCK_C = 4096
_NB = (_COLS + _BLOCK_C - 1) // _BLOCK_C

_U32 = jnp.uint32
_TINY = 1.1754943508222875e-38


def _threefry2x32(x1):
    ks0 = _U32(0)
    ks1 = _U32(42)
    ks2 = _U32(0 ^ 42 ^ 0x1BD11BDA)

    def rotl(x, d):
        return (x << _U32(d)) | (x >> _U32(32 - d))

    def rounds(x0, x1, rots):
        for r in rots:
            x0 = x0 + x1
            x1 = rotl(x1, r)
            x1 = x0 ^ x1
        return x0, x1

    r_even = (13, 15, 26, 6)
    r_odd = (17, 29, 16, 24)
    x1 = x1 + ks1
    x0 = x1
    x1 = x1 ^ rotl(x1, 13)
    x0, x1 = rounds(x0, x1, r_even[1:])
    x0 = x0 + ks1
    x1 = x1 + ks2 + _U32(1)
    x0, x1 = rounds(x0, x1, r_odd)
    x0 = x0 + ks2
    x1 = x1 + ks0 + _U32(2)
    x0, x1 = rounds(x0, x1, r_even)
    x0 = x0 + ks0
    x1 = x1 + ks1 + _U32(3)
    x0, x1 = rounds(x0, x1, r_odd)
    x0 = x0 + ks1
    x1 = x1 + ks2 + _U32(4)
    x0, x1 = rounds(x0, x1, r_even)
    x0 = x0 + ks2
    x1 = x1 + ks0 + _U32(5)
    return x0, x1


def _sample_kernel(logits_ref, out_ref, acc_ref, idx_ref):
    step = pl.program_id(0)
    col0 = step * _BLOCK_C

    blk = logits_ref[...]
    j = col0 + jax.lax.broadcasted_iota(jnp.int32, blk.shape, 1)
    row = jax.lax.broadcasted_iota(jnp.int32, blk.shape, 0)
    n = (row * _COLS + j).astype(_U32)

    r0, r1 = _threefry2x32(n)
    bits = r0 ^ r1

    fb = (bits >> _U32(9)) | _U32(0x3F800000)
    floats = jax.lax.bitcast_convert_type(fb, jnp.float32) - jnp.float32(1.0)
    u = jnp.maximum(_TINY, floats + _TINY)
    g = -jnp.log(-jnp.log(u))

    val = g + blk
    val = jnp.where(j < _COLS, val, jnp.float32(-jnp.inf))

    acc_old = jnp.where(step == 0, jnp.float32(-jnp.inf), acc_ref[...])
    upd = val > acc_old
    acc_ref[...] = jnp.maximum(val, acc_old)
    idx_ref[...] = jnp.where(upd, j, idx_ref[...])

    @pl.when(step == _NB - 1)
    def _():
        acc = acc_ref[...]
        idx = idx_ref[...]
        bmax = jnp.max(acc, axis=1, keepdims=True)
        cand = jnp.where(acc == bmax, idx, jnp.int32(2**31 - 1))
        out_ref[...] = jnp.min(cand, axis=1, keepdims=True)


@jax.jit
def kernel(logits):
    out = pl.pallas_call(
        _sample_kernel,
        grid=(_NB,),
        in_specs=[
            pl.BlockSpec((_ROWS, _BLOCK_C), lambda i: (0, i)),
        ],
        out_specs=pl.BlockSpec((_ROWS, 1), lambda i: (0, 0)),
        out_shape=jax.ShapeDtypeStruct((_ROWS, 1), jnp.int32),
        scratch_shapes=[
            pltpu.VMEM((_ROWS, _BLOCK_C), jnp.float32),
            pltpu.VMEM((_ROWS, _BLOCK_C), jnp.int32),
        ],
    )(logits)
    return out.reshape(_ROWS).astype(jnp.int64)


# speedup vs baseline: 1.0489x; 1.0167x over previous
import jax
import jax.numpy as jnp
from jax.experimental import pallas as pl
from jax.experimental.pallas import tpu as pltpu

_ROWS = 128
_COLS = 100000
_BLOCK_C = 2048
_NB = (_COLS + _BLOCK_C - 1) // _BLOCK_C

_U32 = jnp.uint32
_TINY = 1.1754943508222875e-38


def _threefry2x32(x1):
    ks0 = _U32(0)
    ks1 = _U32(42)
    ks2 = _U32(0 ^ 42 ^ 0x1BD11BDA)

    def rotl(x, d):
        return (x << _U32(d)) | (x >> _U32(32 - d))

    def rounds(x0, x1, rots):
        for r in rots:
            x0 = x0 + x1
            x1 = rotl(x1, r)
            x1 = x0 ^ x1
        return x0, x1

    r_even = (13, 15, 26, 6)
    r_odd = (17, 29, 16, 24)
    x1 = x1 + ks1
    x0 = x1
    x1 = x1 ^ rotl(x1, 13)
    x0, x1 = rounds(x0, x1, r_even[1:])
    x0 = x0 + ks1
    x1 = x1 + ks2 + _U32(1)
    x0, x1 = rounds(x0, x1, r_odd)
    x0 = x0 + ks2
    x1 = x1 + ks0 + _U32(2)
    x0, x1 = rounds(x0, x1, r_even)
    x0 = x0 + ks0
    x1 = x1 + ks1 + _U32(3)
    x0, x1 = rounds(x0, x1, r_odd)
    x0 = x0 + ks1
    x1 = x1 + ks2 + _U32(4)
    x0, x1 = rounds(x0, x1, r_even)
    x0 = x0 + ks2
    x1 = x1 + ks0 + _U32(5)
    return x0, x1


def _sample_kernel(logits_ref, out_ref, acc_ref, idx_ref):
    step = pl.program_id(0)
    col0 = step * _BLOCK_C

    blk = logits_ref[...]
    j = col0 + jax.lax.broadcasted_iota(jnp.int32, blk.shape, 1)
    row = jax.lax.broadcasted_iota(jnp.int32, blk.shape, 0)
    n = (row * _COLS + j).astype(_U32)

    r0, r1 = _threefry2x32(n)
    bits = r0 ^ r1

    fb = (bits >> _U32(9)) | _U32(0x3F800000)
    floats = jax.lax.bitcast_convert_type(fb, jnp.float32) - jnp.float32(1.0)
    u = jnp.maximum(_TINY, floats + _TINY)
    g = -jnp.log(-jnp.log(u))

    val = g + blk
    val = jnp.where(j < _COLS, val, jnp.float32(-jnp.inf))

    acc_old = jnp.where(step == 0, jnp.float32(-jnp.inf), acc_ref[...])
    upd = val > acc_old
    acc_ref[...] = jnp.maximum(val, acc_old)
    idx_ref[...] = jnp.where(upd, j, idx_ref[...])

    @pl.when(step == _NB - 1)
    def _():
        acc = acc_ref[...]
        idx = idx_ref[...]
        bmax = jnp.max(acc, axis=1, keepdims=True)
        cand = jnp.where(acc == bmax, idx, jnp.int32(2**31 - 1))
        out_ref[...] = jnp.min(cand, axis=1, keepdims=True)


@jax.jit
def kernel(logits):
    out = pl.pallas_call(
        _sample_kernel,
        grid=(_NB,),
        in_specs=[
            pl.BlockSpec((_ROWS, _BLOCK_C), lambda i: (0, i)),
        ],
        out_specs=pl.BlockSpec((_ROWS, 1), lambda i: (0, 0)),
        out_shape=jax.ShapeDtypeStruct((_ROWS, 1), jnp.int32),
        scratch_shapes=[
            pltpu.VMEM((_ROWS, _BLOCK_C), jnp.float32),
            pltpu.VMEM((_ROWS, _BLOCK_C), jnp.int32),
        ],
    )(logits)
    return out.reshape(_ROWS).astype(jnp.int64)
